# baseline (speedup 1.0000x reference)
import functools

import jax
import jax.numpy as jnp
from jax import lax
from jax.experimental import pallas as pl
from jax.experimental.pallas import tpu as pltpu
from jax.experimental.pallas import tpu_sc as plsc

K = 1024
D = 10
DP = 16
N = 8192
BLK = 2048
GRID = N // BLK

_SC_INFO = plsc.get_sparse_core_info()
_NC = _SC_INFO.num_cores
_NS = _SC_INFO.num_subcores
_NW = _NC * _NS
_BPW = N // _NW
_CHUNK = 128
_NCHUNK = _BPW // _CHUNK


def _dist_argmin_kernel(x_ref, w_ref, idx_ref, loss_ref):
    pid = pl.program_id(0)
    x = x_ref[...]
    wt = w_ref[...].T
    acc = jnp.zeros((BLK, K), dtype=jnp.float32)
    for d in range(D):
        diff = x[:, d][:, None] - wt[d, :][None, :]
        acc = acc + diff * diff
    idx = jnp.argmin(acc, axis=1)
    idx_ref[...] = idx.reshape(1, 1, BLK)
    partial = jnp.sum(jnp.min(acc, axis=1)).reshape(1, 1)

    @pl.when(pid == 0)
    def _():
        loss_ref[...] = jnp.zeros((1, 1), jnp.float32)

    loss_ref[...] += partial

    @pl.when(pid == GRID - 1)
    def _():
        loss_ref[...] = loss_ref[...] / (N * D)


def _sc_gather_kernel(table_hbm, idx_hbm, out_hbm, idx_v, rows_v, sem):
    wid = lax.axis_index("s") * _NC + lax.axis_index("c")
    row0 = wid * _NCHUNK
    pltpu.sync_copy(idx_hbm.at[pl.ds(row0, _NCHUNK)], idx_v)
    copies = []
    for j in range(_NCHUNK):
        copies.append(
            pltpu.async_copy(
                table_hbm.at[idx_v.at[j]],
                rows_v.at[pl.ds(j * _CHUNK, _CHUNK)],
                sem))
    for c in copies:
        c.wait()
    pltpu.sync_copy(rows_v, out_hbm.at[pl.ds(wid * _BPW, _BPW)])


@functools.partial(
    pl.kernel,
    mesh=plsc.VectorSubcoreMesh(core_axis_name="c", subcore_axis_name="s"),
    out_type=jax.ShapeDtypeStruct((N, DP), jnp.float32),
    scratch_types=[
        pltpu.VMEM((_NCHUNK, _CHUNK), jnp.int32),
        pltpu.VMEM((_BPW, DP), jnp.float32),
        pltpu.SemaphoreType.DMA,
    ],
    compiler_params=pltpu.CompilerParams(use_tc_tiling_on_sc=False),
)
def _sc_gather(table_hbm, idx_hbm, out_hbm, idx_v, rows_v, sem):
    _sc_gather_kernel(table_hbm, idx_hbm, out_hbm, idx_v, rows_v, sem)


@jax.jit
def kernel(encoder_embedding, embedding_weight):
    idx3, loss = pl.pallas_call(
        _dist_argmin_kernel,
        grid=(GRID,),
        in_specs=[
            pl.BlockSpec((BLK, D), lambda i: (i, 0)),
            pl.BlockSpec((K, D), lambda i: (0, 0)),
        ],
        out_specs=[
            pl.BlockSpec((1, 1, BLK), lambda i: (i, 0, 0)),
            pl.BlockSpec((1, 1), lambda i: (0, 0)),
        ],
        out_shape=[
            jax.ShapeDtypeStruct((GRID, 1, BLK), jnp.int32),
            jax.ShapeDtypeStruct((1, 1), jnp.float32),
        ],
    )(encoder_embedding, embedding_weight)
    idx2 = idx3.reshape(N // _CHUNK, _CHUNK)
    wpad = jnp.pad(embedding_weight, ((0, 0), (0, DP - D)))
    qpad = _sc_gather(wpad, idx2)
    return qpad[:, :D], loss[0, 0]


# speedup vs baseline: 1.0006x; 1.0006x over previous
import functools

import jax
import jax.numpy as jnp
from jax import lax
from jax.experimental import pallas as pl
from jax.experimental.pallas import tpu as pltpu
from jax.experimental.pallas import tpu_sc as plsc

K = 1024
D = 10
DP = 16
N = 8192
BLK = 2048
GRID = N // BLK

_SC_INFO = plsc.get_sparse_core_info()
_NC = _SC_INFO.num_cores
_NS = _SC_INFO.num_subcores
_NW = _NC * _NS
_BPW = N // _NW
_CHUNK = 128
_NCHUNK = _BPW // _CHUNK


def _dist_argmin_kernel(x_ref, w_ref, idx_ref, loss_ref):
    pid = pl.program_id(0)
    x = x_ref[...]
    wt = w_ref[...].T
    acc = jnp.zeros((BLK, K), dtype=jnp.float32)
    for d in range(D):
        diff = x[:, d][:, None] - wt[d, :][None, :]
        acc = acc + diff * diff
    idx = jnp.argmin(acc, axis=1)
    idx_ref[...] = idx.reshape(1, 1, BLK)
    partial = jnp.sum(jnp.min(acc, axis=1)).reshape(1, 1)

    @pl.when(pid == 0)
    def _():
        loss_ref[...] = jnp.zeros((1, 1), jnp.float32)

    loss_ref[...] += partial

    @pl.when(pid == GRID - 1)
    def _():
        loss_ref[...] = loss_ref[...] / (N * D)


def _sc_gather_kernel(table_hbm, idx_hbm, out_hbm, idx_v, rows_v, sem):
    wid = lax.axis_index("s") * _NC + lax.axis_index("c")
    row0 = wid * _NCHUNK
    pltpu.sync_copy(idx_hbm.at[pl.ds(row0, _NCHUNK)], idx_v)
    copies = []
    for j in range(_NCHUNK):
        copies.append(
            pltpu.async_copy(
                table_hbm.at[idx_v.at[j]],
                rows_v.at[pl.ds(j * _CHUNK, _CHUNK)],
                sem))
    for c in copies:
        c.wait()
    pltpu.sync_copy(rows_v, out_hbm.at[pl.ds(wid * _BPW, _BPW)])


@functools.partial(
    pl.kernel,
    mesh=plsc.VectorSubcoreMesh(core_axis_name="c", subcore_axis_name="s"),
    out_type=jax.ShapeDtypeStruct((N, D), jnp.float32),
    scratch_types=[
        pltpu.VMEM((_NCHUNK, _CHUNK), jnp.int32),
        pltpu.VMEM((_BPW, D), jnp.float32),
        pltpu.SemaphoreType.DMA,
    ],
    compiler_params=pltpu.CompilerParams(use_tc_tiling_on_sc=False),
)
def _sc_gather(table_hbm, idx_hbm, out_hbm, idx_v, rows_v, sem):
    _sc_gather_kernel(table_hbm, idx_hbm, out_hbm, idx_v, rows_v, sem)


@jax.jit
def kernel(encoder_embedding, embedding_weight):
    idx3, loss = pl.pallas_call(
        _dist_argmin_kernel,
        grid=(GRID,),
        in_specs=[
            pl.BlockSpec((BLK, D), lambda i: (i, 0)),
            pl.BlockSpec((K, D), lambda i: (0, 0)),
        ],
        out_specs=[
            pl.BlockSpec((1, 1, BLK), lambda i: (i, 0, 0)),
            pl.BlockSpec((1, 1), lambda i: (0, 0)),
        ],
        out_shape=[
            jax.ShapeDtypeStruct((GRID, 1, BLK), jnp.int32),
            jax.ShapeDtypeStruct((1, 1), jnp.float32),
        ],
    )(encoder_embedding, embedding_weight)
    idx2 = idx3.reshape(N // _CHUNK, _CHUNK)
    q = _sc_gather(embedding_weight, idx2)
    return q, loss[0, 0]


# speedup vs baseline: 1.3541x; 1.3533x over previous
import functools

import jax
import jax.numpy as jnp
from jax.experimental import pallas as pl
from jax.experimental.pallas import tpu as pltpu

K = 1024
D = 10
N = 8192
BLK = 2048
GRID = N // BLK


def _vq_kernel(x_ref, w_ref, out_ref, loss_ref):
    pid = pl.program_id(0)
    x = x_ref[...]
    wt = w_ref[...].T
    acc = jnp.zeros((BLK, K), dtype=jnp.float32)
    for d in range(D):
        diff = x[:, d][:, None] - wt[d, :][None, :]
        acc = acc + diff * diff
    idx = jnp.argmin(acc, axis=1)
    onehot = (jax.lax.broadcasted_iota(jnp.int32, (BLK, K), 1)
              == idx[:, None]).astype(jnp.float32)
    q = jax.lax.dot_general(
        onehot, w_ref[...],
        dimension_numbers=(((1,), (0,)), ((), ())),
        preferred_element_type=jnp.float32)
    out_ref[...] = x + (q - x)
    partial = jnp.sum((x - q) ** 2).reshape(1, 1)

    @pl.when(pid == 0)
    def _():
        loss_ref[...] = jnp.zeros((1, 1), jnp.float32)

    loss_ref[...] += partial

    @pl.when(pid == GRID - 1)
    def _():
        loss_ref[...] = loss_ref[...] / (N * D)


@jax.jit
def kernel(encoder_embedding, embedding_weight):
    out, loss = pl.pallas_call(
        _vq_kernel,
        grid=(GRID,),
        in_specs=[
            pl.BlockSpec((BLK, D), lambda i: (i, 0)),
            pl.BlockSpec((K, D), lambda i: (0, 0)),
        ],
        out_specs=[
            pl.BlockSpec((BLK, D), lambda i: (i, 0)),
            pl.BlockSpec((1, 1), lambda i: (0, 0)),
        ],
        out_shape=[
            jax.ShapeDtypeStruct((N, D), jnp.float32),
            jax.ShapeDtypeStruct((1, 1), jnp.float32),
        ],
    )(encoder_embedding, embedding_weight)
    return out, loss[0, 0]
